# baseline (speedup 1.0000x reference)
import functools

import jax
import jax.numpy as jnp
from jax import lax
from jax.experimental import pallas as pl
from jax.experimental.pallas import tpu as pltpu
from jax.experimental.pallas import tpu_sc as plsc

DIM = 128
BATCH = 16384

NC = 2
NS = 16
NW = NC * NS
B_PER_W = BATCH // NW


NUM_HOURS = 24
CH = 128
NCHUNK = B_PER_W // CH


def _make_lookup():
    mesh = plsc.VectorSubcoreMesh(core_axis_name="c", subcore_axis_name="s")

    @functools.partial(
        pl.kernel,
        mesh=mesh,
        out_type=jax.ShapeDtypeStruct((BATCH, DIM), jnp.float32),
        scratch_types=[
            pltpu.VMEM((B_PER_W,), jnp.int32),
            pltpu.VMEM((CH, DIM), jnp.float32),
            pltpu.VMEM((CH, DIM), jnp.float32),
            pltpu.VMEM((CH, DIM), jnp.float32),
            pltpu.VMEM((CH, DIM), jnp.float32),
            pltpu.VMEM_SHARED((NUM_HOURS, DIM), jnp.float32),
            pltpu.SemaphoreType.DMA,
            pltpu.SemaphoreType.DMA,
            pltpu.SemaphoreType.DMA,
            pltpu.SemaphoreType.DMA,
            pltpu.SemaphoreType.DMA,
        ],
    )
    def k(table_hbm, idx_hbm, out_hbm, idx_v, rows0, rows1, rows2, rows3,
          table_sh, g0, g1, g2, g3, osem):
        sid = lax.axis_index("s")
        wid = sid * NC + lax.axis_index("c")
        base = wid * B_PER_W
        @pl.when(sid == 0)
        def _():
            pltpu.sync_copy(table_hbm, table_sh)

        pltpu.sync_copy(idx_hbm.at[pl.ds(base, B_PER_W)], idx_v)
        plsc.subcore_barrier()

        bufs = (rows0, rows1, rows2, rows3)
        gsems = (g0, g1, g2, g3)
        gathers = [
            pltpu.async_copy(
                table_sh.at[idx_v.at[pl.ds(c * CH, CH)]], bufs[c], gsems[c]
            )
            for c in range(NCHUNK)
        ]
        outs = []
        for c in range(NCHUNK):
            gathers[c].wait()
            outs.append(
                pltpu.async_copy(bufs[c], out_hbm.at[pl.ds(base + c * CH, CH)], osem)
            )
        for o in outs:
            o.wait()

    return k


_lookup = _make_lookup()


def kernel(hour, hour_table):
    idx = hour.astype(jnp.int32)
    return _lookup(hour_table, idx)


# speedup vs baseline: 1.0055x; 1.0055x over previous
import functools

import jax
import jax.numpy as jnp
from jax import lax
from jax.experimental import pallas as pl
from jax.experimental.pallas import tpu as pltpu
from jax.experimental.pallas import tpu_sc as plsc

DIM = 128
BATCH = 16384

NC = 2
NS = 16
NW = NC * NS
B_PER_W = BATCH // NW


NUM_HOURS = 24
CH = 256
NCHUNK = B_PER_W // CH


def _make_lookup():
    mesh = plsc.VectorSubcoreMesh(core_axis_name="c", subcore_axis_name="s")

    @functools.partial(
        pl.kernel,
        mesh=mesh,
        out_type=jax.ShapeDtypeStruct((BATCH, DIM), jnp.float32),
        scratch_types=[
            pltpu.VMEM((B_PER_W,), jnp.int32),
            pltpu.VMEM((CH, DIM), jnp.float32),
            pltpu.VMEM((CH, DIM), jnp.float32),
            pltpu.VMEM_SHARED((NUM_HOURS, DIM), jnp.float32),
            pltpu.SemaphoreType.DMA,
            pltpu.SemaphoreType.DMA,
            pltpu.SemaphoreType.DMA,
            pltpu.SemaphoreType.DMA,
        ],
    )
    def k(table_hbm, idx_hbm, out_hbm, idx_v, rows0, rows1, table_sh, g0, g1, o0, o1):
        sid = lax.axis_index("s")
        wid = sid * NC + lax.axis_index("c")
        base = wid * B_PER_W
        @pl.when(sid == 0)
        def _():
            pltpu.sync_copy(table_hbm, table_sh)

        pltpu.sync_copy(idx_hbm.at[pl.ds(base, B_PER_W)], idx_v)
        plsc.subcore_barrier()

        bufs = (rows0, rows1)
        gsems = (g0, g1)
        osems = (o0, o1)
        gathers = [None] * NCHUNK
        outs = [None] * NCHUNK
        for c in range(NCHUNK):
            b = c % 2
            if c >= 2:
                outs[c - 2].wait()
            gathers[c] = pltpu.async_copy(
                table_sh.at[idx_v.at[pl.ds(c * CH, CH)]], bufs[b], gsems[b]
            )
            gathers[c].wait()
            outs[c] = pltpu.async_copy(
                bufs[b], out_hbm.at[pl.ds(base + c * CH, CH)], osems[b]
            )
        outs[NCHUNK - 2].wait()
        outs[NCHUNK - 1].wait()

    return k


_lookup = _make_lookup()


def kernel(hour, hour_table):
    idx = hour.astype(jnp.int32)
    return _lookup(hour_table, idx)


# speedup vs baseline: 1.0238x; 1.0182x over previous
import functools

import jax
import jax.numpy as jnp
from jax import lax
from jax.experimental import pallas as pl
from jax.experimental.pallas import tpu as pltpu
from jax.experimental.pallas import tpu_sc as plsc

DIM = 128
BATCH = 16384

NC = 2
NS = 16
NW = NC * NS
B_PER_W = BATCH // NW


NUM_HOURS = 24
CH = 256
NCHUNK = B_PER_W // CH


def _make_lookup():
    mesh = plsc.VectorSubcoreMesh(core_axis_name="c", subcore_axis_name="s")

    @functools.partial(
        pl.kernel,
        mesh=mesh,
        out_type=jax.ShapeDtypeStruct((BATCH, DIM), jnp.float32),
        scratch_types=[
            pltpu.VMEM((B_PER_W,), jnp.int32),
            pltpu.VMEM((CH, DIM), jnp.float32),
            pltpu.VMEM((CH, DIM), jnp.float32),
            pltpu.VMEM_SHARED((NUM_HOURS, DIM), jnp.float32),
            pltpu.SemaphoreType.DMA,
            pltpu.SemaphoreType.DMA,
            pltpu.SemaphoreType.DMA,
            pltpu.SemaphoreType.DMA,
            pltpu.SemaphoreType.DMA,
            pltpu.SemaphoreType.DMA,
        ],
    )
    def k(table_hbm, idx_hbm, out_hbm, idx_v, rows0, rows1, table_sh,
          g0, g1, o0, o1, i0, i1):
        sid = lax.axis_index("s")
        wid = sid * NC + lax.axis_index("c")
        base = wid * B_PER_W
        isems = (i0, i1)
        idx_copies = [
            pltpu.async_copy(
                idx_hbm.at[pl.ds(base + c * CH, CH)],
                idx_v.at[pl.ds(c * CH, CH)],
                isems[c],
            )
            for c in range(NCHUNK)
        ]
        @pl.when(sid == 0)
        def _():
            pltpu.sync_copy(table_hbm, table_sh)

        plsc.subcore_barrier()

        bufs = (rows0, rows1)
        gsems = (g0, g1)
        osems = (o0, o1)
        gathers = [None] * NCHUNK
        outs = [None] * NCHUNK
        for c in range(NCHUNK):
            b = c % 2
            if c >= 2:
                outs[c - 2].wait()
            idx_copies[c].wait()
            gathers[c] = pltpu.async_copy(
                table_sh.at[idx_v.at[pl.ds(c * CH, CH)]], bufs[b], gsems[b]
            )
            gathers[c].wait()
            outs[c] = pltpu.async_copy(
                bufs[b], out_hbm.at[pl.ds(base + c * CH, CH)], osems[b]
            )
        outs[NCHUNK - 2].wait()
        outs[NCHUNK - 1].wait()

    return k


_lookup = _make_lookup()


def kernel(hour, hour_table):
    idx = hour.astype(jnp.int32)
    return _lookup(hour_table, idx)
